# Initial kernel scaffold; baseline (speedup 1.0000x reference)
import jax
import jax.numpy as jnp
from jax.experimental import pallas as pl


def kernel(x, edge_index, edge_attr, lin_W, lin_b, e1_W, e1_b, e2_W, e2_b, reg1_W, reg1_b, reg2_W, reg2_b, mean_W, mean_b, std_W, std_b, cls1_W, cls1_b, cls2_W, cls2_b):
    raise NotImplementedError("write your pallas kernel here")


# speedup vs baseline: 2.9579x; 2.9579x over previous
import functools

import jax
import jax.numpy as jnp
from jax import lax
from jax.experimental import pallas as pl
from jax.experimental.pallas import tpu as pltpu
from jax.experimental.pallas import tpu_sc as plsc

_N = 10000
_E = 320000
_D = 128
_NPAD = 10240
_CHUNK = 128
_NSC = 2
_NSUB = 16
_CPW = 79
_EPAD = _NSC * _NSUB * _CPW * _CHUNK
_ROWS_PER_SUB = _NPAD // _NSUB
_ZR = 64
_BN = 2000
_BE = 4096


def _sc_scatter_layer(hlin, ew, src, dst):
    mesh = plsc.VectorSubcoreMesh(core_axis_name="c", subcore_axis_name="s")

    @functools.partial(
        pl.kernel,
        out_type=jax.ShapeDtypeStruct((_NSC, _NPAD, _D), jnp.float32),
        mesh=mesh,
        scratch_types=[
            pltpu.VMEM((_CHUNK,), jnp.int32),
            pltpu.VMEM((_CHUNK,), jnp.int32),
            pltpu.VMEM((_CHUNK,), jnp.float32),
            pltpu.VMEM((_CHUNK, _D), jnp.float32),
            pltpu.VMEM((_ZR, _D), jnp.float32),
            pltpu.VMEM_SHARED((_NPAD, _D), jnp.float32),
            pltpu.SemaphoreType.DMA,
        ],
    )
    def sc(hlin_hbm, ew_hbm, src_hbm, dst_hbm, out_hbm,
           src_v, dst_v, ew_v, rows_v, zero_v, accum, sem):
        c = lax.axis_index("c")
        s = lax.axis_index("s")
        wid = c * _NSUB + s

        def _zfill(r, carry):
            for q in range(_D // 16):
                zero_v[r, pl.ds(q * 16, 16)] = jnp.zeros((16,), jnp.float32)
            return carry
        lax.fori_loop(0, _ZR, _zfill, 0)

        def _zcopy(b, carry):
            pltpu.sync_copy(zero_v,
                            accum.at[pl.ds(s * _ROWS_PER_SUB + b * _ZR, _ZR)])
            return carry
        lax.fori_loop(0, _ROWS_PER_SUB // _ZR, _zcopy, 0)
        plsc.subcore_barrier()

        def _chunk(t, carry):
            off = (wid * _CPW + t) * _CHUNK
            pltpu.sync_copy(src_hbm.at[pl.ds(off, _CHUNK)], src_v)
            pltpu.sync_copy(dst_hbm.at[pl.ds(off, _CHUNK)], dst_v)
            pltpu.sync_copy(ew_hbm.at[pl.ds(off, _CHUNK)], ew_v)
            pltpu.async_copy(hlin_hbm.at[src_v], rows_v, sem).wait()

            def _scale(g, carry2):
                evec = ew_v[pl.ds(g * 16, 16)]
                for m in range(16):
                    sv = jnp.full((16,), evec[m], jnp.float32)
                    j = g * 16 + m
                    for q in range(_D // 16):
                        rows_v[j, pl.ds(q * 16, 16)] = (
                            rows_v[j, pl.ds(q * 16, 16)] * sv)
                return carry2
            lax.fori_loop(0, _CHUNK // 16, _scale, 0)

            pltpu.sync_copy(rows_v, accum.at[dst_v], add=True)
            return carry
        lax.fori_loop(0, _CPW, _chunk, 0)
        plsc.subcore_barrier()

        pltpu.sync_copy(accum.at[pl.ds(s * _ROWS_PER_SUB, _ROWS_PER_SUB)],
                        out_hbm.at[c, pl.ds(s * _ROWS_PER_SUB, _ROWS_PER_SUB)])

    return sc(hlin, ew, src, dst)


def _ew_body(attrT_ref, w1_ref, b1_ref, w2_ref, b2_ref, out_ref):
    a = attrT_ref[...]
    for l in range(4):
        w1 = w1_ref[l]
        h1 = jnp.dot(w1, a, preferred_element_type=jnp.float32)
        h1 = jnp.maximum(h1 + b1_ref[:, l:l + 1], 0.0)
        w2 = w2_ref[l:l + 1, :]
        z = jnp.dot(w2, h1, preferred_element_type=jnp.float32)
        z = z + b2_ref[l, 0]
        out_ref[pl.ds(l, 1), :] = jax.nn.sigmoid(z)


def _edge_weights(attrT, e1_wt, e1_bt, e2_w, e2_b):
    grid = _EPAD // _BE
    return pl.pallas_call(
        _ew_body,
        grid=(grid,),
        in_specs=[
            pl.BlockSpec((8, _BE), lambda i: (0, i)),
            pl.BlockSpec((4, 16, 8), lambda i: (0, 0, 0)),
            pl.BlockSpec((16, 8), lambda i: (0, 0)),
            pl.BlockSpec((8, 16), lambda i: (0, 0)),
            pl.BlockSpec(memory_space=pltpu.SMEM),
        ],
        out_specs=pl.BlockSpec((8, _BE), lambda i: (0, i)),
        out_shape=jax.ShapeDtypeStruct((8, _EPAD), jnp.float32),
    )(attrT, e1_wt, e1_bt, e2_w, e2_b)


def _lin0_body(x_ref, w_ref, b_ref, out_ref):
    out_ref[...] = (
        jnp.dot(x_ref[...], w_ref[...], preferred_element_type=jnp.float32)
        + b_ref[...])


def _lin0(x, w, b):
    return pl.pallas_call(
        _lin0_body,
        grid=(_N // _BN,),
        in_specs=[
            pl.BlockSpec((_BN, _D), lambda i: (i, 0)),
            pl.BlockSpec((_D, _D), lambda i: (0, 0)),
            pl.BlockSpec((1, _D), lambda i: (0, 0)),
        ],
        out_specs=pl.BlockSpec((_BN, _D), lambda i: (i, 0)),
        out_shape=jax.ShapeDtypeStruct((_N, _D), jnp.float32),
    )(x, w, b)


def _fuse_body(p0_ref, p1_ref, w_ref, b_ref, out_ref):
    h = jnp.maximum(p0_ref[0] + p1_ref[0], 0.0)
    out_ref[...] = (
        jnp.dot(h, w_ref[...], preferred_element_type=jnp.float32)
        + b_ref[...])


def _fuse(part, w, b):
    return pl.pallas_call(
        _fuse_body,
        grid=(_N // _BN,),
        in_specs=[
            pl.BlockSpec((1, _BN, _D), lambda i: (0, i, 0)),
            pl.BlockSpec((1, _BN, _D), lambda i: (1, i, 0)),
            pl.BlockSpec((_D, _D), lambda i: (0, 0)),
            pl.BlockSpec((1, _D), lambda i: (0, 0)),
        ],
        out_specs=pl.BlockSpec((_BN, _D), lambda i: (i, 0)),
        out_shape=jax.ShapeDtypeStruct((_N, _D), jnp.float32),
    )(part, part, w, b)


def _head_body(p0_ref, p1_ref, r1w_ref, r1b_ref, r2w_ref, r2b_ref,
               mw_ref, mb_ref, sw_ref, sb_ref, c1w_ref, c1b_ref,
               c2w_ref, c2b_ref, m_ref, s_ref, l_ref):
    h = jnp.maximum(p0_ref[0] + p1_ref[0], 0.0)
    r1 = jnp.maximum(
        jnp.dot(h, r1w_ref[...], preferred_element_type=jnp.float32)
        + r1b_ref[...], 0.0)
    reg = jnp.maximum(
        jnp.dot(r1, r2w_ref[...], preferred_element_type=jnp.float32)
        + r2b_ref[...], 0.0)
    m_ref[...] = (
        jnp.dot(reg, mw_ref[...], preferred_element_type=jnp.float32)
        + mb_ref[...])
    s_ref[...] = jax.nn.softplus(
        jnp.dot(reg, sw_ref[...], preferred_element_type=jnp.float32)
        + sb_ref[...])
    c1 = jnp.maximum(
        jnp.dot(h, c1w_ref[...], preferred_element_type=jnp.float32)
        + c1b_ref[...], 0.0)
    l_ref[...] = (
        jnp.dot(c1, c2w_ref[...], preferred_element_type=jnp.float32)
        + c2b_ref[...])


def _head(part, r1w, r1b, r2w, r2b, mw, mb, sw, sb, c1w, c1b, c2w, c2b):
    small = lambda shape: pl.BlockSpec(shape, lambda i: tuple(0 for _ in shape))
    return pl.pallas_call(
        _head_body,
        grid=(_N // _BN,),
        in_specs=[
            pl.BlockSpec((1, _BN, _D), lambda i: (0, i, 0)),
            pl.BlockSpec((1, _BN, _D), lambda i: (1, i, 0)),
            small((_D, 64)), small((1, 64)),
            small((64, 32)), small((1, 32)),
            small((32, 8)), small((1, 8)),
            small((32, 8)), small((1, 8)),
            small((_D, 64)), small((1, 64)),
            small((64, 8)), small((1, 8)),
        ],
        out_specs=[
            pl.BlockSpec((_BN, 8), lambda i: (i, 0)),
            pl.BlockSpec((_BN, 8), lambda i: (i, 0)),
            pl.BlockSpec((_BN, 8), lambda i: (i, 0)),
        ],
        out_shape=[
            jax.ShapeDtypeStruct((_N, 8), jnp.float32),
            jax.ShapeDtypeStruct((_N, 8), jnp.float32),
            jax.ShapeDtypeStruct((_N, 8), jnp.float32),
        ],
    )(part, part, r1w, r1b, r2w, r2b, mw, mb, sw, sb, c1w, c1b, c2w, c2b)


def kernel(x, edge_index, edge_attr, lin_W, lin_b, e1_W, e1_b, e2_W, e2_b,
           reg1_W, reg1_b, reg2_W, reg2_b, mean_W, mean_b, std_W, std_b,
           cls1_W, cls1_b, cls2_W, cls2_b):
    pad = _EPAD - _E
    src = jnp.concatenate([edge_index[0], jnp.zeros((pad,), jnp.int32)])
    dst = jnp.concatenate([edge_index[1], jnp.full((pad,), _N, jnp.int32)])

    attrT = jnp.pad(edge_attr.T, ((0, 4), (0, pad)))
    e1_wt = jnp.pad(jnp.swapaxes(e1_W, 1, 2), ((0, 0), (0, 0), (0, 4)))
    e1_bt = jnp.pad(e1_b.T, ((0, 0), (0, 4)))
    e2_w = jnp.pad(e2_W[:, :, 0], ((0, 4), (0, 0)))
    ew8 = _edge_weights(attrT, e1_wt, e1_bt, e2_w, e2_b)

    r1b = reg1_b.reshape(1, 64)
    r2b = reg2_b.reshape(1, 32)
    mw = jnp.pad(mean_W, ((0, 0), (0, 7)))
    mb = jnp.pad(mean_b, (0, 7)).reshape(1, 8)
    sw = jnp.pad(std_W, ((0, 0), (0, 7)))
    sb = jnp.pad(std_b, (0, 7)).reshape(1, 8)
    c1b = cls1_b.reshape(1, 64)
    c2w = jnp.pad(cls2_W, ((0, 0), (0, 6)))
    c2b = jnp.pad(cls2_b, (0, 6)).reshape(1, 8)

    hlin = _lin0(x, lin_W[0], lin_b[0].reshape(1, _D))
    part = None
    for i in range(4):
        part = _sc_scatter_layer(hlin, ew8[i], src, dst)
        if i < 3:
            hlin = _fuse(part, lin_W[i + 1], lin_b[i + 1].reshape(1, _D))

    m8, s8, l8 = _head(part, reg1_W, r1b, reg2_W, r2b, mw, mb, sw, sb,
                       cls1_W, c1b, c2w, c2b)
    return m8[:, 0], s8[:, 0], l8[:, :2]
